# Initial kernel scaffold; baseline (speedup 1.0000x reference)
import jax
import jax.numpy as jnp
from jax.experimental import pallas as pl


def kernel(ind, labels, num_ood):
    raise NotImplementedError("write your pallas kernel here")


# speedup vs baseline: 1.2794x; 1.2794x over previous
import functools

import numpy as np
import jax
import jax.numpy as jnp
from jax import lax
from jax.experimental import pallas as pl
from jax.experimental.pallas import tpu as pltpu
from jax.experimental.pallas import tpu_sc as plsc

_NUM_OOD = 1024
_SEL = 2
_D = 128
_LANES = 16
_NC = 2
_NS = 16
_NW = _NC * _NS
_ROWS_PER_W = _NUM_OOD // _NW
_IDX_PER_W = _ROWS_PER_W * _SEL


def _mixture_plan(labels_np, num_ood):
    rng = np.random.RandomState(0)
    label_set = np.unique(labels_np)
    cdt_all = []
    s_all = []
    while len(cdt_all) < num_ood:
        select_number = _SEL
        select_label = rng.choice(label_set, select_number, replace=False)
        cdt = []
        for label in select_label:
            idx = np.where(labels_np == label)[0]
            cdt.append(int(rng.choice(idx, 1)[0]))
        s = rng.dirichlet(alpha=[1.0] * select_number)
        cdt_all.append(cdt)
        s_all.append(s)
    cdt_arr = np.asarray(cdt_all, dtype=np.int32)
    s_arr = np.asarray(s_all, dtype=np.float32)
    return cdt_arr, s_arr


def _plan_cb(labels_v, num_ood_v):
    cdt_arr, s_arr = _mixture_plan(np.asarray(labels_v), int(num_ood_v))
    return (np.asarray(cdt_arr, dtype=np.int32),
            np.asarray(s_arr, dtype=np.float32))


_CHUNKS = 4
_ROWS_PER_CHUNK = _ROWS_PER_W // _CHUNKS
_IDX_PER_CHUNK = _ROWS_PER_CHUNK * _SEL


def _ood_mix_body(ind_hbm, cdt_hbm, s_hbm, out_hbm, idx_v, w_v,
                  rows0, rows1, rows2, rows3, out0, out1, out2, out3,
                  sem_i, sem_w, sem_g0, sem_g1, sem_g2, sem_g3,
                  sem_o0, sem_o1, sem_o2, sem_o3):
    rows_bufs = (rows0, rows1, rows2, rows3)
    out_bufs = (out0, out1, out2, out3)
    gather_sems = (sem_g0, sem_g1, sem_g2, sem_g3)
    out_sems = (sem_o0, sem_o1, sem_o2, sem_o3)

    wid = lax.axis_index("s") * _NC + lax.axis_index("c")
    base = wid * _IDX_PER_W
    h_idx = pltpu.async_copy(cdt_hbm.at[pl.ds(base, _IDX_PER_W)], idx_v, sem_i)
    h_w = pltpu.async_copy(s_hbm.at[pl.ds(base, _IDX_PER_W)], w_v, sem_w)
    h_idx.wait()
    h_rows = [
        pltpu.async_copy(
            ind_hbm.at[idx_v.at[pl.ds(c * _IDX_PER_CHUNK, _IDX_PER_CHUNK)]],
            rows_bufs[c], gather_sems[c])
        for c in range(_CHUNKS)
    ]
    h_w.wait()

    h_out = []
    for c in range(_CHUNKS):
        h_rows[c].wait()
        rows_v = rows_bufs[c]
        out_v = out_bufs[c]
        wv = w_v[pl.ds(c * _IDX_PER_CHUNK, _LANES)]
        for r in range(_ROWS_PER_CHUNK):
            w0 = wv.at[jnp.full((_LANES,), 2 * r, jnp.int32)].get(
                mode="promise_in_bounds")
            w1 = wv.at[jnp.full((_LANES,), 2 * r + 1, jnp.int32)].get(
                mode="promise_in_bounds")
            for j in range(_D // _LANES):
                sl = pl.ds(j * _LANES, _LANES)
                out_v[r, sl] = (w0 * rows_v[2 * r, sl]
                                + w1 * rows_v[2 * r + 1, sl])
        h_out.append(pltpu.async_copy(
            out_v,
            out_hbm.at[pl.ds(wid * _ROWS_PER_W + c * _ROWS_PER_CHUNK,
                             _ROWS_PER_CHUNK)],
            out_sems[c]))
    for h in h_out:
        h.wait()


@functools.cache
def _build_ood_mix_kernel():
    return pl.kernel(
        _ood_mix_body,
        mesh=plsc.VectorSubcoreMesh(core_axis_name="c", subcore_axis_name="s"),
        out_type=jax.ShapeDtypeStruct((_NUM_OOD, _D), jnp.float32),
        scratch_types=(
            [pltpu.VMEM((_IDX_PER_W,), jnp.int32),
             pltpu.VMEM((_IDX_PER_W,), jnp.float32)]
            + [pltpu.VMEM((_IDX_PER_CHUNK, _D), jnp.float32)
               for _ in range(_CHUNKS)]
            + [pltpu.VMEM((_ROWS_PER_CHUNK, _D), jnp.float32)
               for _ in range(_CHUNKS)]
            + [pltpu.SemaphoreType.DMA for _ in range(2 + 2 * _CHUNKS)]
        ),
    )


def kernel(ind, labels, num_ood):
    cdt, s = jax.pure_callback(
        _plan_cb,
        (
            jax.ShapeDtypeStruct((_NUM_OOD, _SEL), jnp.int32),
            jax.ShapeDtypeStruct((_NUM_OOD, _SEL), jnp.float32),
        ),
        labels,
        num_ood,
    )
    return _build_ood_mix_kernel()(ind, cdt.reshape(-1), s.reshape(-1))
